# Initial kernel scaffold; baseline (speedup 1.0000x reference)
import jax
import jax.numpy as jnp
from jax.experimental import pallas as pl


def kernel(adapter_feat, caption_embed, logit_scale, v2p_map, caption_to_point_mapping, caption_idx):
    raise NotImplementedError("write your pallas kernel here")


# speedup vs baseline: 1.4288x; 1.4288x over previous
import functools

import jax
import jax.numpy as jnp
from jax import lax
from jax.experimental import pallas as pl
from jax.experimental.pallas import tpu as pltpu
from jax.experimental.pallas import tpu_sc as plsc

NC = 2
NS = 16
NW = NC * NS
L = 16
CHUNK = 128
NCAP = 10000
NCAP_PAD = 10240
ROWS_PER_SUB = NCAP_PAD // NS
D = 128
CROWS = 128


def _round_up(x, m):
    return (x + m - 1) // m * m


def _make_gather_kernel(n_pad, d):
    rows_per_w = n_pad // NW
    n_chunks = rows_per_w // CHUNK
    mesh = plsc.VectorSubcoreMesh(core_axis_name="c", subcore_axis_name="s")

    @functools.partial(
        pl.kernel,
        out_type=jax.ShapeDtypeStruct((n_pad, d), jnp.float32),
        mesh=mesh,
        scratch_types=[
            pltpu.VMEM((1, CHUNK), jnp.int32),
            pltpu.VMEM((CHUNK, d), jnp.float32),
            pltpu.SemaphoreType.DMA,
        ],
    )
    def gather_rows(table_hbm, idx_hbm, out_hbm, idx_v, rows_v, sem):
        wid = lax.axis_index("s") * NC + lax.axis_index("c")
        base = wid * rows_per_w

        def body(j, carry):
            off = base + j * CHUNK
            pltpu.sync_copy(idx_hbm.at[pl.ds(off, CHUNK)], idx_v.at[0])
            pltpu.async_copy(table_hbm.at[idx_v.at[0]], rows_v, sem).wait()
            pltpu.sync_copy(rows_v, out_hbm.at[pl.ds(off, CHUNK)])
            return carry

        lax.fori_loop(0, n_chunks, body, 0)

    return gather_rows


def _make_scores_kernel(n_pad, d, c, blk):
    def body(feats_ref, ce_ref, scale_ref, out_ref):
        x = feats_ref[...]
        nrm = jnp.sqrt(jnp.sum(x * x, axis=1, keepdims=True))
        nrm = jnp.maximum(nrm, 1e-12)
        xn = x / nrm
        scale = jnp.exp(scale_ref[0])
        logits = lax.dot_general(
            xn, ce_ref[...], (((1,), (1,)), ((), ())),
            preferred_element_type=jnp.float32,
        ) * scale
        mx = jnp.max(logits, axis=1, keepdims=True)
        lse = jnp.log(jnp.sum(jnp.exp(logits - mx), axis=1, keepdims=True)) + mx
        out_ref[...] = logits - lse

    grid = (n_pad // blk,)
    return pl.pallas_call(
        body,
        grid=grid,
        in_specs=[
            pl.BlockSpec((blk, d), lambda i: (i, 0)),
            pl.BlockSpec((c, d), lambda i: (0, 0)),
            pl.BlockSpec(memory_space=pltpu.MemorySpace.SMEM),
        ],
        out_specs=pl.BlockSpec((blk, c), lambda i: (i, 0)),
        out_shape=jax.ShapeDtypeStruct((n_pad, c), jnp.float32),
    )


def _make_segment_kernel(n_pad, m_pad):
    pairs_per_w = m_pad // NW
    n_rounds = pairs_per_w // CHUNK
    crows_per_sub = CROWS // NS
    mesh = plsc.VectorSubcoreMesh(core_axis_name="c", subcore_axis_name="s")

    @functools.partial(
        pl.kernel,
        out_type=(
            jax.ShapeDtypeStruct((NC, NCAP_PAD, D), jnp.float32),
            jax.ShapeDtypeStruct((NC, CROWS, D), jnp.float32),
        ),
        mesh=mesh,
        scratch_types=[
            pltpu.VMEM((1, CHUNK), jnp.int32),
            pltpu.VMEM((1, CHUNK), jnp.int32),
            pltpu.VMEM((1, CHUNK), jnp.int32),
            pltpu.VMEM((1, CHUNK), jnp.int32),
            pltpu.VMEM((CHUNK, D), jnp.float32),
            pltpu.VMEM((CHUNK, D), jnp.float32),
            pltpu.VMEM_SHARED((NCAP_PAD, D), jnp.float32),
            pltpu.VMEM_SHARED((CROWS, D), jnp.float32),
            pltpu.SemaphoreType.DMA,
        ],
    )
    def segment_sum(scores_hbm, ctp_hbm, cap_hbm, eye_hbm, zrow_hbm,
                    pooled_hbm, cnt_hbm,
                    ctp_v, cap_v, cmod_v, cdiv_v, rows_v, ones_v,
                    acc, cacc, sem):
        cid = lax.axis_index("c")
        sid = lax.axis_index("s")
        wid = sid * NC + cid
        sl = pl.ds(sid * ROWS_PER_SUB, ROWS_PER_SUB)
        csl = pl.ds(sid * crows_per_sub, crows_per_sub)
        pltpu.sync_copy(zrow_hbm, acc.at[sl])
        pltpu.sync_copy(zrow_hbm.at[pl.ds(0, crows_per_sub)], cacc.at[csl])
        plsc.subcore_barrier()

        base = wid * pairs_per_w

        def round_body(r, carry):
            off = base + r * CHUNK
            pltpu.sync_copy(ctp_hbm.at[pl.ds(off, CHUNK)], ctp_v.at[0])
            pltpu.sync_copy(cap_hbm.at[pl.ds(off, CHUNK)], cap_v.at[0])
            for t in range(CHUNK // L):
                ts = pl.ds(t * L, L)
                cv = cap_v[0, ts]
                cmod_v[0, ts] = jnp.bitwise_and(cv, D - 1)
                cdiv_v[0, ts] = jax.lax.shift_right_logical(cv, 7)
            pltpu.async_copy(scores_hbm.at[ctp_v.at[0]], rows_v, sem).wait()
            pltpu.async_copy(eye_hbm.at[cmod_v.at[0]], ones_v, sem).wait()
            pltpu.sync_copy(rows_v, acc.at[cap_v.at[0]], add=True)
            pltpu.sync_copy(ones_v, cacc.at[cdiv_v.at[0]], add=True)
            return carry

        lax.fori_loop(0, n_rounds, round_body, 0)
        plsc.subcore_barrier()
        pltpu.sync_copy(acc.at[sl], pooled_hbm.at[cid, sl])
        pltpu.sync_copy(cacc.at[csl], cnt_hbm.at[cid, csl])

    return segment_sum


def _make_counts_kernel():
    def body(cc_ref, out_ref):
        out_ref[...] = cc_ref[0] + cc_ref[1]

    return pl.pallas_call(
        body,
        in_specs=[pl.BlockSpec((NC, CROWS, D), lambda: (0, 0, 0))],
        out_specs=pl.BlockSpec((CROWS, D), lambda: (0, 0)),
        out_shape=jax.ShapeDtypeStruct((CROWS, D), jnp.float32),
    )


def _make_finalize_kernel(blk):
    def body(pp_ref, cnt_ref, out_ref, cnt_out_ref):
        p = pp_ref[0] + pp_ref[1]
        c0 = cnt_ref[...]
        cnt_out_ref[...] = c0
        safe = jnp.where(c0 > 0, c0, 1.0)
        denom = jnp.where(c0 > 0, 1.0 / safe, 0.0)
        out_ref[...] = p * denom

    grid = (NCAP_PAD // blk,)
    return pl.pallas_call(
        body,
        grid=grid,
        in_specs=[
            pl.BlockSpec((NC, blk, D), lambda i: (0, i, 0)),
            pl.BlockSpec((blk, 1), lambda i: (i, 0)),
        ],
        out_specs=[
            pl.BlockSpec((blk, D), lambda i: (i, 0)),
            pl.BlockSpec((blk, 1), lambda i: (i, 0)),
        ],
        out_shape=[
            jax.ShapeDtypeStruct((NCAP_PAD, D), jnp.float32),
            jax.ShapeDtypeStruct((NCAP_PAD, 1), jnp.float32),
        ],
    )


@jax.jit
def kernel(adapter_feat, caption_embed, logit_scale, v2p_map,
           caption_to_point_mapping, caption_idx):
    v, d = adapter_feat.shape
    c = caption_embed.shape[0]
    n = v2p_map.shape[0]
    m = caption_to_point_mapping.shape[0]

    n_pad = _round_up(n, NW * CHUNK)
    m_pad = _round_up(m, NW * CHUNK)

    v2p_pad = jnp.concatenate(
        [v2p_map.astype(jnp.int32), jnp.zeros((n_pad - n,), jnp.int32)])
    ctp_pad = jnp.concatenate(
        [caption_to_point_mapping.astype(jnp.int32),
         jnp.zeros((m_pad - m,), jnp.int32)])
    cap_pad = jnp.concatenate(
        [caption_idx.astype(jnp.int32),
         jnp.full((m_pad - m,), NCAP, jnp.int32)])

    feats = _make_gather_kernel(n_pad, d)(adapter_feat, v2p_pad)

    scale_arr = jnp.reshape(logit_scale.astype(jnp.float32), (1,))
    scores = _make_scores_kernel(n_pad, d, c, 512)(feats, caption_embed,
                                                   scale_arr)

    eye = jnp.eye(D, dtype=jnp.float32)
    zrow = jnp.zeros((ROWS_PER_SUB, D), jnp.float32)
    pooled_part, cnt_part = _make_segment_kernel(n_pad, m_pad)(
        scores, ctp_pad, cap_pad, eye, zrow)

    cnt_mat = _make_counts_kernel()(cnt_part)
    cnt_col = jnp.reshape(cnt_mat, (CROWS * D,))[:NCAP_PAD, None]
    pooled, counts = _make_finalize_kernel(2560)(pooled_part, cnt_col)
    return pooled[:NCAP], counts[:NCAP, 0]
